# baseline (speedup 1.0000x reference)
import functools

import jax
import jax.numpy as jnp
from jax import lax
from jax.experimental import pallas as pl
from jax.experimental.pallas import tpu as pltpu
from jax.experimental.pallas import tpu_sc as plsc

N_NODES = 10000
H_DIM = 128
OUT_DIM = 64
N_GRAPH = 128
N_EDGE = 320000
N_LAYER = 3

NUM_CORES = 2
NUM_SUBCORES = 16
NW = NUM_CORES * NUM_SUBCORES
EPW = N_EDGE // NW
CHUNK = 128
NBUF = 2
NGROUP = 39
EPW_MAIN = NGROUP * NBUF * CHUNK
TAIL = EPW - EPW_MAIN
ROWS_PER_TILE = 624

_sc_mesh = plsc.VectorSubcoreMesh(core_axis_name="c", subcore_axis_name="s")


@functools.partial(
    pl.kernel,
    out_type=jax.ShapeDtypeStruct((NUM_CORES, N_NODES, H_DIM), jnp.float32),
    mesh=_sc_mesh,
    scratch_types=[
        pltpu.VMEM((2, NBUF, CHUNK), jnp.int32),
        pltpu.VMEM((2, NBUF, CHUNK), jnp.int32),
        pltpu.VMEM((2, TAIL), jnp.int32),
        [pltpu.VMEM((CHUNK, H_DIM), jnp.float32) for _ in range(NBUF)],
        pltpu.VMEM((16, H_DIM), jnp.float32),
        pltpu.VMEM_SHARED((N_NODES, H_DIM), jnp.float32),
        [pltpu.SemaphoreType.DMA for _ in range(2)],
        [pltpu.SemaphoreType.DMA for _ in range(NBUF)],
        [pltpu.SemaphoreType.DMA for _ in range(NBUF)],
        pltpu.SemaphoreType.DMA,
    ],
)
def _sc_agg(h_hbm, src_hbm, dst_hbm, tsrc_hbm, tdst_hbm, out_hbm,
            srcg, dstg, tidx, rows, zero_v, agg_s, isem, gsem, ssem, zsem):
    cid = lax.axis_index("c")
    sid = lax.axis_index("s")
    wid = cid * NUM_SUBCORES + sid

    def start_i(g, p):
        pltpu.async_copy(src_hbm.at[wid, g], srcg.at[p], isem[p])
        pltpu.async_copy(dst_hbm.at[wid, g], dstg.at[p], isem[p])

    def wait_i(p):
        pltpu.make_async_copy(src_hbm.at[0, 0], srcg.at[p], isem[p]).wait()
        pltpu.make_async_copy(dst_hbm.at[0, 0], dstg.at[p], isem[p]).wait()

    def start_g(b, p):
        pltpu.async_copy(h_hbm.at[srcg.at[p, b]], rows[b], gsem[b])

    def wait_g(b):
        pltpu.make_async_copy(h_hbm.at[srcg.at[0, 0]], rows[b], gsem[b]).wait()

    def start_s(b, p):
        pltpu.async_copy(rows[b], agg_s.at[dstg.at[p, b]], ssem[b], add=True)

    def wait_s(b):
        pltpu.make_async_copy(rows[b], agg_s.at[dstg.at[0, 0]], ssem[b]).wait()

    start_i(0, 0)
    start_i(1, 1)
    pltpu.async_copy(tsrc_hbm.at[wid], tidx.at[0], isem[0])
    pltpu.async_copy(tdst_hbm.at[wid], tidx.at[1], isem[0])

    z16 = jnp.zeros((16,), jnp.float32)
    for r in range(16):
        for c in range(H_DIM // 16):
            zero_v[r, pl.ds(c * 16, 16)] = z16

    nzero = jnp.where(sid == NUM_SUBCORES - 1, 40, 39)

    def zero_body(i, carry):
        pltpu.async_copy(zero_v, agg_s.at[pl.ds(sid * ROWS_PER_TILE + i * 16, 16)],
                         zsem)
        return carry

    lax.fori_loop(0, nzero, zero_body, 0)

    pltpu.make_async_copy(tsrc_hbm.at[0], tidx.at[0], isem[0]).wait()
    pltpu.make_async_copy(tdst_hbm.at[0], tidx.at[1], isem[0]).wait()
    wait_i(0)
    for b in range(NBUF):
        start_g(b, 0)

    def zero_drain(i, carry):
        pltpu.make_async_copy(zero_v, agg_s.at[pl.ds(sid * ROWS_PER_TILE, 16)],
                              zsem).wait()
        return carry

    lax.fori_loop(0, nzero, zero_drain, 0)
    plsc.subcore_barrier()


    def do_group(g, p, q, refill, has_next):
        for b in range(NBUF):
            wait_g(b)
            start_s(b, p)
        if has_next:
            wait_i(q)
        for b in range(NBUF):
            wait_s(b)
            if has_next:
                start_g(b, q)
        if refill:
            start_i(g + 2, p)

    def body(tt, carry):
        g0 = 2 * tt
        do_group(g0, 0, 1, True, True)
        do_group(g0 + 1, 1, 0, True, True)
        return carry

    lax.fori_loop(0, NGROUP // 2 - 1, body, 0)
    do_group(NGROUP - 3, 0, 1, True, True)
    do_group(NGROUP - 2, 1, 0, False, True)
    do_group(NGROUP - 1, 0, 1, False, False)

    pltpu.async_copy(h_hbm.at[tidx.at[0]], rows[0].at[pl.ds(0, TAIL)], gsem[0])
    pltpu.make_async_copy(h_hbm.at[tidx.at[0]], rows[0].at[pl.ds(0, TAIL)],
                          gsem[0]).wait()
    pltpu.sync_copy(rows[0].at[pl.ds(0, TAIL)], agg_s.at[tidx.at[1]], add=True)
    plsc.subcore_barrier()

    pltpu.sync_copy(
        agg_s.at[pl.ds(sid * ROWS_PER_TILE, ROWS_PER_TILE)],
        out_hbm.at[cid, pl.ds(sid * ROWS_PER_TILE, ROWS_PER_TILE)],
    )

    @pl.when(sid == NUM_SUBCORES - 1)
    def _():
        tail = NUM_SUBCORES * ROWS_PER_TILE
        pltpu.sync_copy(
            agg_s.at[pl.ds(tail, N_NODES - tail)],
            out_hbm.at[cid, pl.ds(tail, N_NODES - tail)],
        )


BN = 1000
NB = N_NODES // BN


def _mlp_body(hin_ref, p_ref, w1_ref, b1_ref, gamma_ref, beta_ref,
              w2_ref, b2_ref, batch_ref, wo_ref, bo_ref, sin_ref,
              h_ref, contrib_ref, z_s, ssum_s, ssq_s, pooled_s):
    ph = pl.program_id(0)
    j = pl.program_id(1)

    @pl.when(ph == 0)
    def _():
        y = hin_ref[...] + p_ref[0] + p_ref[1]
        z = jnp.dot(y, w1_ref[...], preferred_element_type=jnp.float32) + b1_ref[...]
        z_s[pl.ds(j * BN, BN), :] = z

        @pl.when(j == 0)
        def _():
            ssum_s[...] = jnp.zeros_like(ssum_s)
            ssq_s[...] = jnp.zeros_like(ssq_s)

        ssum_s[...] += jnp.sum(z, axis=0, keepdims=True)
        ssq_s[...] += jnp.sum(z * z, axis=0, keepdims=True)

    @pl.when(ph == 1)
    def _():
        m = ssum_s[...] * (1.0 / N_NODES)
        v = ssq_s[...] * (1.0 / N_NODES) - m * m
        scale = lax.rsqrt(v + 1e-5) * gamma_ref[...]
        zn = (z_s[pl.ds(j * BN, BN), :] - m) * scale + beta_ref[...]
        zn = jnp.maximum(zn, 0.0)
        h = jnp.dot(zn, w2_ref[...], preferred_element_type=jnp.float32) + b2_ref[...]
        h_ref[...] = h

        seg = batch_ref[0]
        g_iota = lax.broadcasted_iota(jnp.int32, (N_GRAPH, BN), 0)
        onehot = (g_iota == seg).astype(jnp.float32)

        @pl.when(j == 0)
        def _():
            pooled_s[...] = jnp.zeros_like(pooled_s)

        pooled_s[...] += jnp.dot(onehot, h, preferred_element_type=jnp.float32)

        @pl.when(j == NB - 1)
        def _():
            contrib_ref[...] = (
                jnp.dot(pooled_s[...], wo_ref[...],
                        preferred_element_type=jnp.float32)
                + bo_ref[...] + sin_ref[...]
            )


def _mlp(hin, pagg, w1, b1, gamma, beta, w2, b2, batch3d, wo, bo, score_in):
    return pl.pallas_call(
        _mlp_body,
        grid=(2, NB),
        in_specs=[
            pl.BlockSpec((BN, H_DIM), lambda p, j: (j * (1 - p), 0)),
            pl.BlockSpec((2, BN, H_DIM), lambda p, j: (0, j * (1 - p), 0)),
            pl.BlockSpec((H_DIM, H_DIM), lambda p, j: (0, 0)),
            pl.BlockSpec((1, H_DIM), lambda p, j: (0, 0)),
            pl.BlockSpec((1, H_DIM), lambda p, j: (0, 0)),
            pl.BlockSpec((1, H_DIM), lambda p, j: (0, 0)),
            pl.BlockSpec((H_DIM, H_DIM), lambda p, j: (0, 0)),
            pl.BlockSpec((1, H_DIM), lambda p, j: (0, 0)),
            pl.BlockSpec((1, 1, BN), lambda p, j: (j, 0, 0)),
            pl.BlockSpec((H_DIM, OUT_DIM), lambda p, j: (0, 0)),
            pl.BlockSpec((1, OUT_DIM), lambda p, j: (0, 0)),
            pl.BlockSpec((N_GRAPH, OUT_DIM), lambda p, j: (0, 0)),
        ],
        out_specs=[
            pl.BlockSpec((BN, H_DIM), lambda p, j: (j * p, 0)),
            pl.BlockSpec((N_GRAPH, OUT_DIM), lambda p, j: (0, 0)),
        ],
        out_shape=[
            jax.ShapeDtypeStruct((N_NODES, H_DIM), jnp.float32),
            jax.ShapeDtypeStruct((N_GRAPH, OUT_DIM), jnp.float32),
        ],
        scratch_shapes=[
            pltpu.VMEM((N_NODES, H_DIM), jnp.float32),
            pltpu.VMEM((1, H_DIM), jnp.float32),
            pltpu.VMEM((1, H_DIM), jnp.float32),
            pltpu.VMEM((N_GRAPH, H_DIM), jnp.float32),
        ],
    )(hin, pagg, w1, b1, gamma, beta, w2, b2, batch3d, wo, bo, score_in)


def kernel(x, edge_index, batch, W1, b1, gamma, beta, W2, b2, Wo, bo):
    ei = edge_index.reshape(2, NW, EPW)
    src = ei[0, :, :EPW_MAIN].reshape(NW, NGROUP, NBUF, CHUNK)
    dst = ei[1, :, :EPW_MAIN].reshape(NW, NGROUP, NBUF, CHUNK)
    tsrc = ei[0, :, EPW_MAIN:]
    tdst = ei[1, :, EPW_MAIN:]
    batch3d = batch.reshape(NB, 1, BN)

    h = x
    score = jnp.zeros((N_GRAPH, OUT_DIM), jnp.float32)
    for l in range(N_LAYER):
        p = _sc_agg(h, src, dst, tsrc, tdst)
        h, score = _mlp(h, p, W1[l], b1[l].reshape(1, H_DIM),
                        gamma[l].reshape(1, H_DIM), beta[l].reshape(1, H_DIM),
                        W2[l], b2[l].reshape(1, H_DIM),
                        batch3d, Wo[l], bo[l].reshape(1, OUT_DIM), score)
    return score


# speedup vs baseline: 1.1508x; 1.1508x over previous
import functools

import jax
import jax.numpy as jnp
from jax import lax
from jax.experimental import pallas as pl
from jax.experimental.pallas import tpu as pltpu
from jax.experimental.pallas import tpu_sc as plsc

N_NODES = 10000
H_DIM = 128
OUT_DIM = 64
N_GRAPH = 128
N_EDGE = 320000
N_LAYER = 3

NUM_CORES = 2
NUM_SUBCORES = 16
NW = NUM_CORES * NUM_SUBCORES
EPW = N_EDGE // NW
CHUNK = 40
NCHUNK = EPW // CHUNK
ROWS_PER_TILE = 624

NBUF = 5
NGROUP = NCHUNK // NBUF

_sc_mesh = plsc.VectorSubcoreMesh(core_axis_name="c", subcore_axis_name="s")


@functools.partial(
    pl.kernel,
    out_type=jax.ShapeDtypeStruct((NUM_CORES, N_NODES, H_DIM), jnp.float32),
    mesh=_sc_mesh,
    scratch_types=[
        pltpu.VMEM((2, NBUF, CHUNK), jnp.int32),
        pltpu.VMEM((2, NBUF, CHUNK), jnp.int32),
        [pltpu.VMEM((CHUNK, H_DIM), jnp.float32) for _ in range(NBUF)],
        pltpu.VMEM((16, H_DIM), jnp.float32),
        pltpu.VMEM_SHARED((N_NODES, H_DIM), jnp.float32),
        [pltpu.SemaphoreType.DMA for _ in range(2)],
        [pltpu.SemaphoreType.DMA for _ in range(NBUF)],
        [pltpu.SemaphoreType.DMA for _ in range(NBUF)],
        pltpu.SemaphoreType.DMA,
    ],
)
def _sc_agg(h_hbm, src_hbm, dst_hbm, out_hbm,
            srcg, dstg, rows, zero_v, agg_s, isem, gsem, ssem, zsem):
    cid = lax.axis_index("c")
    sid = lax.axis_index("s")
    wid = cid * NUM_SUBCORES + sid

    def start_i(g, p):
        pltpu.async_copy(src_hbm.at[wid, g], srcg.at[p], isem[p])
        pltpu.async_copy(dst_hbm.at[wid, g], dstg.at[p], isem[p])

    def wait_i(p):
        pltpu.make_async_copy(src_hbm.at[0, 0], srcg.at[p], isem[p]).wait()
        pltpu.make_async_copy(dst_hbm.at[0, 0], dstg.at[p], isem[p]).wait()

    def start_g(b, p):
        pltpu.async_copy(h_hbm.at[srcg.at[p, b]], rows[b], gsem[b])

    def wait_g(b):
        pltpu.make_async_copy(h_hbm.at[srcg.at[0, 0]], rows[b], gsem[b]).wait()

    def start_s(b, p):
        pltpu.async_copy(rows[b], agg_s.at[dstg.at[p, b]], ssem[b], add=True)

    def wait_s(b):
        pltpu.make_async_copy(rows[b], agg_s.at[dstg.at[0, 0]], ssem[b]).wait()

    start_i(0, 0)
    start_i(1, 1)

    z16 = jnp.zeros((16,), jnp.float32)
    for r in range(16):
        for c in range(H_DIM // 16):
            zero_v[r, pl.ds(c * 16, 16)] = z16

    nzero = jnp.where(sid == NUM_SUBCORES - 1, 40, 39)

    def zero_body(i, carry):
        pltpu.async_copy(zero_v, agg_s.at[pl.ds(sid * ROWS_PER_TILE + i * 16, 16)],
                         zsem)
        return carry

    lax.fori_loop(0, nzero, zero_body, 0)

    wait_i(0)
    for b in range(NBUF):
        start_g(b, 0)

    def zero_drain(i, carry):
        pltpu.make_async_copy(zero_v, agg_s.at[pl.ds(sid * ROWS_PER_TILE, 16)],
                              zsem).wait()
        return carry

    lax.fori_loop(0, nzero, zero_drain, 0)
    plsc.subcore_barrier()


    def do_group(g, p, q, refill, has_next):
        for b in range(NBUF):
            wait_g(b)
            start_s(b, p)
        if has_next:
            wait_i(q)
        for b in range(NBUF):
            wait_s(b)
            if has_next:
                start_g(b, q)
        if refill:
            start_i(g + 2, p)

    def body(tt, carry):
        g0 = 2 * tt
        do_group(g0, 0, 1, True, True)
        do_group(g0 + 1, 1, 0, True, True)
        return carry

    lax.fori_loop(0, NGROUP // 2 - 1, body, 0)
    do_group(NGROUP - 2, 0, 1, False, True)
    do_group(NGROUP - 1, 1, 0, False, False)
    plsc.subcore_barrier()

    pltpu.sync_copy(
        agg_s.at[pl.ds(sid * ROWS_PER_TILE, ROWS_PER_TILE)],
        out_hbm.at[cid, pl.ds(sid * ROWS_PER_TILE, ROWS_PER_TILE)],
    )

    @pl.when(sid == NUM_SUBCORES - 1)
    def _():
        tail = NUM_SUBCORES * ROWS_PER_TILE
        pltpu.sync_copy(
            agg_s.at[pl.ds(tail, N_NODES - tail)],
            out_hbm.at[cid, pl.ds(tail, N_NODES - tail)],
        )


BN = 1000
NB = N_NODES // BN


def _mlp_body(hin_ref, p_ref, w1_ref, b1_ref, gamma_ref, beta_ref,
              w2_ref, b2_ref, batch_ref, wo_ref, bo_ref, sin_ref,
              h_ref, contrib_ref, z_s, ssum_s, ssq_s, pooled_s):
    ph = pl.program_id(0)
    j = pl.program_id(1)

    @pl.when(ph == 0)
    def _():
        y = hin_ref[...] + p_ref[0] + p_ref[1]
        z = jnp.dot(y, w1_ref[...], preferred_element_type=jnp.float32) + b1_ref[...]
        z_s[pl.ds(j * BN, BN), :] = z

        @pl.when(j == 0)
        def _():
            ssum_s[...] = jnp.zeros_like(ssum_s)
            ssq_s[...] = jnp.zeros_like(ssq_s)

        ssum_s[...] += jnp.sum(z, axis=0, keepdims=True)
        ssq_s[...] += jnp.sum(z * z, axis=0, keepdims=True)

    @pl.when(ph == 1)
    def _():
        m = ssum_s[...] * (1.0 / N_NODES)
        v = ssq_s[...] * (1.0 / N_NODES) - m * m
        scale = lax.rsqrt(v + 1e-5) * gamma_ref[...]
        zn = (z_s[pl.ds(j * BN, BN), :] - m) * scale + beta_ref[...]
        zn = jnp.maximum(zn, 0.0)
        h = jnp.dot(zn, w2_ref[...], preferred_element_type=jnp.float32) + b2_ref[...]
        h_ref[...] = h

        seg = batch_ref[0]
        g_iota = lax.broadcasted_iota(jnp.int32, (N_GRAPH, BN), 0)
        onehot = (g_iota == seg).astype(jnp.float32)

        @pl.when(j == 0)
        def _():
            pooled_s[...] = jnp.zeros_like(pooled_s)

        pooled_s[...] += jnp.dot(onehot, h, preferred_element_type=jnp.float32)

        @pl.when(j == NB - 1)
        def _():
            contrib_ref[...] = (
                jnp.dot(pooled_s[...], wo_ref[...],
                        preferred_element_type=jnp.float32)
                + bo_ref[...] + sin_ref[...]
            )


def _mlp(hin, pagg, w1, b1, gamma, beta, w2, b2, batch3d, wo, bo, score_in):
    return pl.pallas_call(
        _mlp_body,
        grid=(2, NB),
        in_specs=[
            pl.BlockSpec((BN, H_DIM), lambda p, j: (j * (1 - p), 0)),
            pl.BlockSpec((2, BN, H_DIM), lambda p, j: (0, j * (1 - p), 0)),
            pl.BlockSpec((H_DIM, H_DIM), lambda p, j: (0, 0)),
            pl.BlockSpec((1, H_DIM), lambda p, j: (0, 0)),
            pl.BlockSpec((1, H_DIM), lambda p, j: (0, 0)),
            pl.BlockSpec((1, H_DIM), lambda p, j: (0, 0)),
            pl.BlockSpec((H_DIM, H_DIM), lambda p, j: (0, 0)),
            pl.BlockSpec((1, H_DIM), lambda p, j: (0, 0)),
            pl.BlockSpec((1, 1, BN), lambda p, j: (j, 0, 0)),
            pl.BlockSpec((H_DIM, OUT_DIM), lambda p, j: (0, 0)),
            pl.BlockSpec((1, OUT_DIM), lambda p, j: (0, 0)),
            pl.BlockSpec((N_GRAPH, OUT_DIM), lambda p, j: (0, 0)),
        ],
        out_specs=[
            pl.BlockSpec((BN, H_DIM), lambda p, j: (j * p, 0)),
            pl.BlockSpec((N_GRAPH, OUT_DIM), lambda p, j: (0, 0)),
        ],
        out_shape=[
            jax.ShapeDtypeStruct((N_NODES, H_DIM), jnp.float32),
            jax.ShapeDtypeStruct((N_GRAPH, OUT_DIM), jnp.float32),
        ],
        scratch_shapes=[
            pltpu.VMEM((N_NODES, H_DIM), jnp.float32),
            pltpu.VMEM((1, H_DIM), jnp.float32),
            pltpu.VMEM((1, H_DIM), jnp.float32),
            pltpu.VMEM((N_GRAPH, H_DIM), jnp.float32),
        ],
    )(hin, pagg, w1, b1, gamma, beta, w2, b2, batch3d, wo, bo, score_in)


def kernel(x, edge_index, batch, W1, b1, gamma, beta, W2, b2, Wo, bo):
    src = edge_index[0].reshape(NW, NGROUP, NBUF, CHUNK)
    dst = edge_index[1].reshape(NW, NGROUP, NBUF, CHUNK)
    batch3d = batch.reshape(NB, 1, BN)

    h = x
    score = jnp.zeros((N_GRAPH, OUT_DIM), jnp.float32)
    for l in range(N_LAYER):
        p = _sc_agg(h, src, dst)
        h, score = _mlp(h, p, W1[l], b1[l].reshape(1, H_DIM),
                        gamma[l].reshape(1, H_DIM), beta[l].reshape(1, H_DIM),
                        W2[l], b2[l].reshape(1, H_DIM),
                        batch3d, Wo[l], bo[l].reshape(1, OUT_DIM), score)
    return score


# speedup vs baseline: 1.2007x; 1.0433x over previous
import functools

import jax
import jax.numpy as jnp
from jax import lax
from jax.experimental import pallas as pl
from jax.experimental.pallas import tpu as pltpu
from jax.experimental.pallas import tpu_sc as plsc

N_NODES = 10000
H_DIM = 128
OUT_DIM = 64
N_GRAPH = 128
N_EDGE = 320000
N_LAYER = 3

NUM_CORES = 2
NUM_SUBCORES = 16
NW = NUM_CORES * NUM_SUBCORES
EPW = N_EDGE // NW
CHUNK = 40
NCHUNK = EPW // CHUNK
ROWS_PER_TILE = 624

NBUF = 5
NGROUP = NCHUNK // NBUF

_sc_mesh = plsc.VectorSubcoreMesh(core_axis_name="c", subcore_axis_name="s")


@functools.partial(
    pl.kernel,
    out_type=jax.ShapeDtypeStruct((NUM_CORES, N_NODES, H_DIM), jnp.float32),
    mesh=_sc_mesh,
    scratch_types=[
        pltpu.VMEM((2, NBUF, CHUNK), jnp.int32),
        pltpu.VMEM((2, NBUF, CHUNK), jnp.int32),
        [pltpu.VMEM((CHUNK, H_DIM), jnp.float32) for _ in range(NBUF)],
        pltpu.VMEM((16, H_DIM), jnp.float32),
        pltpu.VMEM_SHARED((N_NODES, H_DIM), jnp.float32),
        [pltpu.SemaphoreType.DMA for _ in range(2)],
        [pltpu.SemaphoreType.DMA for _ in range(NBUF)],
        [pltpu.SemaphoreType.DMA for _ in range(NBUF)],
        pltpu.SemaphoreType.DMA,
    ],
)
def _sc_agg(h_hbm, src_hbm, dst_hbm, out_hbm,
            srcg, dstg, rows, zero_v, agg_s, isem, gsem, ssem, zsem):
    cid = lax.axis_index("c")
    sid = lax.axis_index("s")
    wid = cid * NUM_SUBCORES + sid

    def start_i(g, p):
        pltpu.async_copy(src_hbm.at[wid, g], srcg.at[p], isem[p])
        pltpu.async_copy(dst_hbm.at[wid, g], dstg.at[p], isem[p])

    def wait_i(p):
        pltpu.make_async_copy(src_hbm.at[0, 0], srcg.at[p], isem[p]).wait()
        pltpu.make_async_copy(dst_hbm.at[0, 0], dstg.at[p], isem[p]).wait()

    def start_g(b, p):
        pltpu.async_copy(h_hbm.at[srcg.at[p, b]], rows[b], gsem[b])

    def wait_g(b):
        pltpu.make_async_copy(h_hbm.at[srcg.at[0, 0]], rows[b], gsem[b]).wait()

    def start_s(b, p):
        pltpu.async_copy(rows[b], agg_s.at[dstg.at[p, b]], ssem[b], add=True)

    def wait_s(b):
        pltpu.make_async_copy(rows[b], agg_s.at[dstg.at[0, 0]], ssem[b]).wait()

    start_i(0, 0)
    start_i(1, 1)

    z16 = jnp.zeros((16,), jnp.float32)
    for r in range(16):
        for c in range(H_DIM // 16):
            zero_v[r, pl.ds(c * 16, 16)] = z16

    nzero = jnp.where(sid == NUM_SUBCORES - 1, 40, 39)

    def zero_body(i, carry):
        pltpu.async_copy(zero_v, agg_s.at[pl.ds(sid * ROWS_PER_TILE + i * 16, 16)],
                         zsem)
        return carry

    lax.fori_loop(0, nzero, zero_body, 0)

    wait_i(0)
    for b in range(NBUF):
        start_g(b, 0)

    def zero_drain(i, carry):
        pltpu.make_async_copy(zero_v, agg_s.at[pl.ds(sid * ROWS_PER_TILE, 16)],
                              zsem).wait()
        return carry

    lax.fori_loop(0, nzero, zero_drain, 0)
    plsc.subcore_barrier()


    def do_group(g, p, q, refill, has_next):
        for b in range(NBUF):
            wait_g(b)
            start_s(b, p)
        if has_next:
            wait_i(q)
        for b in range(NBUF):
            wait_s(b)
            if has_next:
                start_g(b, q)
        if refill:
            start_i(g + 2, p)

    def body(tt, carry):
        g0 = 2 * tt
        do_group(g0, 0, 1, True, True)
        do_group(g0 + 1, 1, 0, True, True)
        return carry

    lax.fori_loop(0, NGROUP // 2 - 1, body, 0)
    do_group(NGROUP - 2, 0, 1, False, True)
    do_group(NGROUP - 1, 1, 0, False, False)
    plsc.subcore_barrier()

    pltpu.sync_copy(
        agg_s.at[pl.ds(sid * ROWS_PER_TILE, ROWS_PER_TILE)],
        out_hbm.at[cid, pl.ds(sid * ROWS_PER_TILE, ROWS_PER_TILE)],
    )

    @pl.when(sid == NUM_SUBCORES - 1)
    def _():
        tail = NUM_SUBCORES * ROWS_PER_TILE
        pltpu.sync_copy(
            agg_s.at[pl.ds(tail, N_NODES - tail)],
            out_hbm.at[cid, pl.ds(tail, N_NODES - tail)],
        )


BN = 2000
NB = N_NODES // BN


def _mlp_body(hin_ref, p_ref, w1_ref, b1_ref, gamma_ref, beta_ref,
              w2_ref, b2_ref, batch_ref, wo_ref, bo_ref, sin_ref,
              h_ref, contrib_ref, z_s, ssum_s, ssq_s, pooled_s):
    ph = pl.program_id(0)
    j = pl.program_id(1)

    @pl.when(ph == 0)
    def _():
        y = hin_ref[...] + p_ref[0] + p_ref[1]
        z = jnp.dot(y, w1_ref[...], preferred_element_type=jnp.float32) + b1_ref[...]
        z_s[pl.ds(j * BN, BN), :] = z

        @pl.when(j == 0)
        def _():
            ssum_s[...] = jnp.zeros_like(ssum_s)
            ssq_s[...] = jnp.zeros_like(ssq_s)

        ssum_s[...] += jnp.sum(z, axis=0, keepdims=True)
        ssq_s[...] += jnp.sum(z * z, axis=0, keepdims=True)

    @pl.when(ph == 1)
    def _():
        m = ssum_s[...] * (1.0 / N_NODES)
        v = ssq_s[...] * (1.0 / N_NODES) - m * m
        scale = lax.rsqrt(v + 1e-5) * gamma_ref[...]
        zn = (z_s[pl.ds(j * BN, BN), :] - m) * scale + beta_ref[...]
        zn = jnp.maximum(zn, 0.0)
        h = jnp.dot(zn, w2_ref[...], preferred_element_type=jnp.float32) + b2_ref[...]
        h_ref[...] = h

        seg = batch_ref[0]
        g_iota = lax.broadcasted_iota(jnp.int32, (N_GRAPH, BN), 0)
        onehot = (g_iota == seg).astype(jnp.float32)

        @pl.when(j == 0)
        def _():
            pooled_s[...] = jnp.zeros_like(pooled_s)

        pooled_s[...] += jnp.dot(onehot, h, preferred_element_type=jnp.float32)

        @pl.when(j == NB - 1)
        def _():
            contrib_ref[...] = (
                jnp.dot(pooled_s[...], wo_ref[...],
                        preferred_element_type=jnp.float32)
                + bo_ref[...] + sin_ref[...]
            )


def _mlp(hin, pagg, w1, b1, gamma, beta, w2, b2, batch3d, wo, bo, score_in):
    return pl.pallas_call(
        _mlp_body,
        grid=(2, NB),
        in_specs=[
            pl.BlockSpec((BN, H_DIM), lambda p, j: (j * (1 - p), 0)),
            pl.BlockSpec((2, BN, H_DIM), lambda p, j: (0, j * (1 - p), 0)),
            pl.BlockSpec((H_DIM, H_DIM), lambda p, j: (0, 0)),
            pl.BlockSpec((1, H_DIM), lambda p, j: (0, 0)),
            pl.BlockSpec((1, H_DIM), lambda p, j: (0, 0)),
            pl.BlockSpec((1, H_DIM), lambda p, j: (0, 0)),
            pl.BlockSpec((H_DIM, H_DIM), lambda p, j: (0, 0)),
            pl.BlockSpec((1, H_DIM), lambda p, j: (0, 0)),
            pl.BlockSpec((1, 1, BN), lambda p, j: (j, 0, 0)),
            pl.BlockSpec((H_DIM, OUT_DIM), lambda p, j: (0, 0)),
            pl.BlockSpec((1, OUT_DIM), lambda p, j: (0, 0)),
            pl.BlockSpec((N_GRAPH, OUT_DIM), lambda p, j: (0, 0)),
        ],
        out_specs=[
            pl.BlockSpec((BN, H_DIM), lambda p, j: (j * p, 0)),
            pl.BlockSpec((N_GRAPH, OUT_DIM), lambda p, j: (0, 0)),
        ],
        out_shape=[
            jax.ShapeDtypeStruct((N_NODES, H_DIM), jnp.float32),
            jax.ShapeDtypeStruct((N_GRAPH, OUT_DIM), jnp.float32),
        ],
        scratch_shapes=[
            pltpu.VMEM((N_NODES, H_DIM), jnp.float32),
            pltpu.VMEM((1, H_DIM), jnp.float32),
            pltpu.VMEM((1, H_DIM), jnp.float32),
            pltpu.VMEM((N_GRAPH, H_DIM), jnp.float32),
        ],
    )(hin, pagg, w1, b1, gamma, beta, w2, b2, batch3d, wo, bo, score_in)


def kernel(x, edge_index, batch, W1, b1, gamma, beta, W2, b2, Wo, bo):
    src = edge_index[0].reshape(NW, NGROUP, NBUF, CHUNK)
    dst = edge_index[1].reshape(NW, NGROUP, NBUF, CHUNK)
    batch3d = batch.reshape(NB, 1, BN)

    h = x
    score = jnp.zeros((N_GRAPH, OUT_DIM), jnp.float32)
    for l in range(N_LAYER):
        p = _sc_agg(h, src, dst)
        h, score = _mlp(h, p, W1[l], b1[l].reshape(1, H_DIM),
                        gamma[l].reshape(1, H_DIM), beta[l].reshape(1, H_DIM),
                        W2[l], b2[l].reshape(1, H_DIM),
                        batch3d, Wo[l], bo[l].reshape(1, OUT_DIM), score)
    return score


# speedup vs baseline: 1.2036x; 1.0025x over previous
import functools

import jax
import jax.numpy as jnp
from jax import lax
from jax.experimental import pallas as pl
from jax.experimental.pallas import tpu as pltpu
from jax.experimental.pallas import tpu_sc as plsc

N_NODES = 10000
H_DIM = 128
OUT_DIM = 64
N_GRAPH = 128
N_EDGE = 320000
N_LAYER = 3

NUM_CORES = 2
NUM_SUBCORES = 16
NW = NUM_CORES * NUM_SUBCORES
EPW = N_EDGE // NW
CHUNK = 40
NCHUNK = EPW // CHUNK
ROWS_PER_TILE = 624

NBUF = 5
NGROUP = NCHUNK // NBUF

_sc_mesh = plsc.VectorSubcoreMesh(core_axis_name="c", subcore_axis_name="s")


@functools.partial(
    pl.kernel,
    out_type=jax.ShapeDtypeStruct((NUM_CORES, N_NODES, H_DIM), jnp.float32),
    mesh=_sc_mesh,
    scratch_types=[
        pltpu.VMEM((2, NBUF, CHUNK), jnp.int32),
        pltpu.VMEM((2, NBUF, CHUNK), jnp.int32),
        [pltpu.VMEM((CHUNK, H_DIM), jnp.float32) for _ in range(NBUF)],
        pltpu.VMEM((16, H_DIM), jnp.float32),
        pltpu.VMEM_SHARED((N_NODES, H_DIM), jnp.float32),
        [pltpu.SemaphoreType.DMA for _ in range(2)],
        [pltpu.SemaphoreType.DMA for _ in range(NBUF)],
        [pltpu.SemaphoreType.DMA for _ in range(NBUF)],
        pltpu.SemaphoreType.DMA,
    ],
)
def _sc_agg(h_hbm, src_hbm, dst_hbm, out_hbm,
            srcg, dstg, rows, zero_v, agg_s, isem, gsem, ssem, zsem):
    cid = lax.axis_index("c")
    sid = lax.axis_index("s")
    wid = cid * NUM_SUBCORES + sid

    def start_i(g, p):
        pltpu.async_copy(src_hbm.at[wid, g], srcg.at[p], isem[p])
        pltpu.async_copy(dst_hbm.at[wid, g], dstg.at[p], isem[p])

    def wait_i(p):
        pltpu.make_async_copy(src_hbm.at[0, 0], srcg.at[p], isem[p]).wait()
        pltpu.make_async_copy(dst_hbm.at[0, 0], dstg.at[p], isem[p]).wait()

    def start_g(b, p):
        pltpu.async_copy(h_hbm.at[srcg.at[p, b]], rows[b], gsem[b])

    def wait_g(b):
        pltpu.make_async_copy(h_hbm.at[srcg.at[0, 0]], rows[b], gsem[b]).wait()

    def start_s(b, p):
        pltpu.async_copy(rows[b], agg_s.at[dstg.at[p, b]], ssem[b], add=True)

    def wait_s(b):
        pltpu.make_async_copy(rows[b], agg_s.at[dstg.at[0, 0]], ssem[b]).wait()

    start_i(0, 0)
    start_i(1, 1)

    z16 = jnp.zeros((16,), jnp.float32)
    for r in range(16):
        for c in range(H_DIM // 16):
            zero_v[r, pl.ds(c * 16, 16)] = z16

    nzero = jnp.where(sid == NUM_SUBCORES - 1, 40, 39)

    def zero_body(i, carry):
        pltpu.async_copy(zero_v, agg_s.at[pl.ds(sid * ROWS_PER_TILE + i * 16, 16)],
                         zsem)
        return carry

    lax.fori_loop(0, nzero, zero_body, 0)

    wait_i(0)
    for b in range(NBUF):
        start_g(b, 0)

    def zero_drain(i, carry):
        pltpu.make_async_copy(zero_v, agg_s.at[pl.ds(sid * ROWS_PER_TILE, 16)],
                              zsem).wait()
        return carry

    lax.fori_loop(0, nzero, zero_drain, 0)
    plsc.subcore_barrier()


    def do_group(g, p, q, refill, has_next):
        for b in range(NBUF):
            wait_g(b)
            start_s(b, p)
        if has_next:
            wait_i(q)
        for b in range(NBUF):
            wait_s(b)
            if has_next:
                start_g(b, q)
        if refill:
            start_i(g + 2, p)

    def body(tt, carry):
        g0 = 2 * tt
        do_group(g0, 0, 1, True, True)
        do_group(g0 + 1, 1, 0, True, True)
        return carry

    lax.fori_loop(0, NGROUP // 2 - 1, body, 0)
    do_group(NGROUP - 2, 0, 1, False, True)
    do_group(NGROUP - 1, 1, 0, False, False)
    plsc.subcore_barrier()

    pltpu.sync_copy(
        agg_s.at[pl.ds(sid * ROWS_PER_TILE, ROWS_PER_TILE)],
        out_hbm.at[cid, pl.ds(sid * ROWS_PER_TILE, ROWS_PER_TILE)],
    )

    @pl.when(sid == NUM_SUBCORES - 1)
    def _():
        tail = NUM_SUBCORES * ROWS_PER_TILE
        pltpu.sync_copy(
            agg_s.at[pl.ds(tail, N_NODES - tail)],
            out_hbm.at[cid, pl.ds(tail, N_NODES - tail)],
        )


BN = 5000
NB = N_NODES // BN


def _mlp_body(hin_ref, p_ref, w1_ref, b1_ref, gamma_ref, beta_ref,
              w2_ref, b2_ref, batch_ref, wo_ref, bo_ref, sin_ref,
              h_ref, contrib_ref, z_s, ssum_s, ssq_s, pooled_s):
    ph = pl.program_id(0)
    j = pl.program_id(1)

    @pl.when(ph == 0)
    def _():
        y = hin_ref[...] + p_ref[0] + p_ref[1]
        z = jnp.dot(y, w1_ref[...], preferred_element_type=jnp.float32) + b1_ref[...]
        z_s[pl.ds(j * BN, BN), :] = z

        @pl.when(j == 0)
        def _():
            ssum_s[...] = jnp.zeros_like(ssum_s)
            ssq_s[...] = jnp.zeros_like(ssq_s)

        ssum_s[...] += jnp.sum(z, axis=0, keepdims=True)
        ssq_s[...] += jnp.sum(z * z, axis=0, keepdims=True)

    @pl.when(ph == 1)
    def _():
        m = ssum_s[...] * (1.0 / N_NODES)
        v = ssq_s[...] * (1.0 / N_NODES) - m * m
        scale = lax.rsqrt(v + 1e-5) * gamma_ref[...]
        zn = (z_s[pl.ds(j * BN, BN), :] - m) * scale + beta_ref[...]
        zn = jnp.maximum(zn, 0.0)
        h = jnp.dot(zn, w2_ref[...], preferred_element_type=jnp.float32) + b2_ref[...]
        h_ref[...] = h

        seg = batch_ref[0]
        g_iota = lax.broadcasted_iota(jnp.int32, (N_GRAPH, BN), 0)
        onehot = (g_iota == seg).astype(jnp.float32)

        @pl.when(j == 0)
        def _():
            pooled_s[...] = jnp.zeros_like(pooled_s)

        pooled_s[...] += jnp.dot(onehot, h, preferred_element_type=jnp.float32)

        @pl.when(j == NB - 1)
        def _():
            contrib_ref[...] = (
                jnp.dot(pooled_s[...], wo_ref[...],
                        preferred_element_type=jnp.float32)
                + bo_ref[...] + sin_ref[...]
            )


def _mlp(hin, pagg, w1, b1, gamma, beta, w2, b2, batch3d, wo, bo, score_in):
    return pl.pallas_call(
        _mlp_body,
        grid=(2, NB),
        in_specs=[
            pl.BlockSpec((BN, H_DIM), lambda p, j: (j * (1 - p), 0)),
            pl.BlockSpec((2, BN, H_DIM), lambda p, j: (0, j * (1 - p), 0)),
            pl.BlockSpec((H_DIM, H_DIM), lambda p, j: (0, 0)),
            pl.BlockSpec((1, H_DIM), lambda p, j: (0, 0)),
            pl.BlockSpec((1, H_DIM), lambda p, j: (0, 0)),
            pl.BlockSpec((1, H_DIM), lambda p, j: (0, 0)),
            pl.BlockSpec((H_DIM, H_DIM), lambda p, j: (0, 0)),
            pl.BlockSpec((1, H_DIM), lambda p, j: (0, 0)),
            pl.BlockSpec((1, 1, BN), lambda p, j: (j, 0, 0)),
            pl.BlockSpec((H_DIM, OUT_DIM), lambda p, j: (0, 0)),
            pl.BlockSpec((1, OUT_DIM), lambda p, j: (0, 0)),
            pl.BlockSpec((N_GRAPH, OUT_DIM), lambda p, j: (0, 0)),
        ],
        out_specs=[
            pl.BlockSpec((BN, H_DIM), lambda p, j: (j * p, 0)),
            pl.BlockSpec((N_GRAPH, OUT_DIM), lambda p, j: (0, 0)),
        ],
        out_shape=[
            jax.ShapeDtypeStruct((N_NODES, H_DIM), jnp.float32),
            jax.ShapeDtypeStruct((N_GRAPH, OUT_DIM), jnp.float32),
        ],
        scratch_shapes=[
            pltpu.VMEM((N_NODES, H_DIM), jnp.float32),
            pltpu.VMEM((1, H_DIM), jnp.float32),
            pltpu.VMEM((1, H_DIM), jnp.float32),
            pltpu.VMEM((N_GRAPH, H_DIM), jnp.float32),
        ],
    )(hin, pagg, w1, b1, gamma, beta, w2, b2, batch3d, wo, bo, score_in)


def kernel(x, edge_index, batch, W1, b1, gamma, beta, W2, b2, Wo, bo):
    src = edge_index[0].reshape(NW, NGROUP, NBUF, CHUNK)
    dst = edge_index[1].reshape(NW, NGROUP, NBUF, CHUNK)
    batch3d = batch.reshape(NB, 1, BN)

    h = x
    score = jnp.zeros((N_GRAPH, OUT_DIM), jnp.float32)
    for l in range(N_LAYER):
        p = _sc_agg(h, src, dst)
        h, score = _mlp(h, p, W1[l], b1[l].reshape(1, H_DIM),
                        gamma[l].reshape(1, H_DIM), beta[l].reshape(1, H_DIM),
                        W2[l], b2[l].reshape(1, H_DIM),
                        batch3d, Wo[l], bo[l].reshape(1, OUT_DIM), score)
    return score
